# Initial kernel scaffold; baseline (speedup 1.0000x reference)
import jax
import jax.numpy as jnp
from jax.experimental import pallas as pl


def kernel(hms, pms_map, origin_shapes, pms_stats, u_base, shp_base, exp_base):
    raise NotImplementedError("write your pallas kernel here")


# speedup vs baseline: 5.3403x; 5.3403x over previous
import functools

import numpy as np
import jax
import jax.numpy as jnp
from jax import lax
from jax.experimental import pallas as pl
from jax.experimental.pallas import tpu as pltpu
from jax.experimental.pallas import tpu_sc as plsc

B, H, W = 8, 160, 160
HW = H * W
P = 91
PPAD = 128
K = 1000
KPAD = 1024
NOBJ = 200
KPTS = 68
IOU_THR = 0.5
NEG = -3.0e38

_INTERPRET = False


def _peak_kernel(hm_ref, out_ref):
    m = hm_ref[0]
    ninf = jnp.full((1, W), -jnp.inf, jnp.float32)
    up = jnp.concatenate([m[1:], ninf], axis=0)
    dn = jnp.concatenate([ninf, m[:-1]], axis=0)
    rmax = jnp.maximum(m, jnp.maximum(up, dn))
    ninfc = jnp.full((H, 1), -jnp.inf, jnp.float32)
    lf = jnp.concatenate([rmax[:, 1:], ninfc], axis=1)
    rt = jnp.concatenate([ninfc, rmax[:, :-1]], axis=1)
    hmax = jnp.maximum(rmax, jnp.maximum(lf, rt))
    out_ref[0] = jnp.where(hmax == m, m, 0.0)


def _peak_call(hm):
    return pl.pallas_call(
        _peak_kernel,
        grid=(B,),
        in_specs=[pl.BlockSpec((1, H, W), lambda b: (b, 0, 0))],
        out_specs=pl.BlockSpec((1, H, W), lambda b: (b, 0, 0)),
        out_shape=jax.ShapeDtypeStruct((B, H, W), jnp.float32),
        interpret=_INTERPRET,
    )(hm)


def _search_kernel(keep_ref, tq_ref):
    keys = lax.bitcast_convert_type(keep_ref[...], jnp.int32)

    def body(_, carry):
        lo, hi = carry
        mid = lo + ((hi - lo + 1) >> 1)
        cnt = jnp.sum((keys >= mid).astype(jnp.int32), axis=1, keepdims=True)
        ge = cnt >= K
        return jnp.where(ge, mid, lo), jnp.where(ge, hi, mid - 1)

    lo0 = jnp.zeros((B, 1), jnp.int32)
    hi0 = jnp.full((B, 1), 0x7F800000, jnp.int32)
    t, _ = lax.fori_loop(0, 31, body, (lo0, hi0))
    cnt_gt = jnp.sum((keys > t).astype(jnp.int32), axis=1, keepdims=True)
    quota = K - cnt_gt
    lane = lax.broadcasted_iota(jnp.int32, (B, 128), 1)
    tq_ref[...] = jnp.where(lane < 16, t, jnp.where(lane < 32, quota, 0))


def _search_call(keep_flat):
    return pl.pallas_call(
        _search_kernel,
        in_specs=[pl.BlockSpec((B, HW), lambda: (0, 0))],
        out_specs=pl.BlockSpec((B, 128), lambda: (0, 0)),
        out_shape=jax.ShapeDtypeStruct((B, 128), jnp.int32),
        interpret=_INTERPRET,
    )(keep_flat)


def _compact_kernel(keep_hbm, tq_hbm, pms_hbm, idx_out, score_out, params_out,
                    keys_v, tqt_v, tqq_v, idxbuf, scorebuf, paramsv, sem):
    cid = lax.axis_index("c")
    sid = lax.axis_index("s")
    wid = sid * 2 + cid

    @pl.when(wid < B)
    def _():
        b = wid
        pltpu.sync_copy(keep_hbm.at[b], keys_v)
        pltpu.sync_copy(tq_hbm.at[b, pl.ds(0, 16)], tqt_v)
        pltpu.sync_copy(tq_hbm.at[b, pl.ds(16, 16)], tqq_v)
        t_vec = tqt_v[...]
        q_vec = tqq_v[...]
        cntgt_vec = K - q_vec
        cntgt = jnp.sum(cntgt_vec) // 16
        lane = lax.iota(jnp.int32, 16)

        def body(i, carry):
            ngt, neq = carry
            v = keys_v[pl.ds(i * 16, 16)]
            kk = plsc.bitcast(v, jnp.int32)
            mgt = kk > t_vec
            meq = kk == t_vec
            flat = lane + i * 16
            sgt = jnp.sum(plsc.all_reduce_population_count(mgt)) // 16

            @pl.when(sgt > 0)
            def _():
                plsc.store_compressed(idxbuf.at[pl.ds(ngt, 16)], flat, mask=mgt)
                plsc.store_compressed(scorebuf.at[pl.ds(ngt, 16)], v, mask=mgt)

            pre = plsc.cumsum(meq.astype(jnp.int32))
            take = meq & ((neq + pre) <= q_vec)
            seq = jnp.sum(plsc.all_reduce_population_count(take)) // 16

            @pl.when(seq > 0)
            def _():
                off = cntgt + neq
                plsc.store_compressed(idxbuf.at[pl.ds(off, 16)], flat, mask=take)
                plsc.store_compressed(scorebuf.at[pl.ds(off, 16)], v, mask=take)

            return ngt + sgt, neq + seq

        lax.fori_loop(0, HW // 16, body, (jnp.int32(0), jnp.int32(0)))

        zl = jnp.zeros((16,), jnp.int32)
        zf = jnp.zeros((16,), jnp.float32)
        for off in (K, K + 8):
            idxbuf[pl.ds(off, 16)] = zl
            scorebuf[pl.ds(off, 16)] = zf
        pms_b = pms_hbm.at[b]
        for half in range(2):
            copies = [
                pltpu.async_copy(
                    pms_b.at[idxbuf.at[pl.ds(half * 512 + j * 128, 128)]],
                    paramsv.at[pl.ds(j * 128, 128)], sem)
                for j in range(4)
            ]
            for cp in copies:
                cp.wait()
            pltpu.sync_copy(paramsv, params_out.at[b, pl.ds(half * 512, 512)])
        pltpu.sync_copy(idxbuf.at[pl.ds(0, KPAD)], idx_out.at[b])
        pltpu.sync_copy(scorebuf.at[pl.ds(0, KPAD)], score_out.at[b])


def _compact_call(keep_flat, tq, pms_flat):
    mesh = plsc.VectorSubcoreMesh(core_axis_name="c", subcore_axis_name="s",
                                  num_cores=2, num_subcores=16)
    f = functools.partial(
        pl.kernel,
        out_type=[
            jax.ShapeDtypeStruct((B, KPAD), jnp.int32),
            jax.ShapeDtypeStruct((B, KPAD), jnp.float32),
            jax.ShapeDtypeStruct((B, KPAD, PPAD), jnp.float32),
        ],
        mesh=mesh,
        scratch_types=[
            pltpu.VMEM((HW,), jnp.float32),
            pltpu.VMEM((16,), jnp.int32),
            pltpu.VMEM((16,), jnp.int32),
            pltpu.VMEM((KPAD + 16,), jnp.int32),
            pltpu.VMEM((KPAD + 16,), jnp.float32),
            pltpu.VMEM((512, PPAD), jnp.float32),
            pltpu.SemaphoreType.DMA,
        ],
        compiler_params=pltpu.CompilerParams(needs_layout_passes=False,
                                             use_tc_tiling_on_sc=False),
        interpret=_INTERPRET,
    )
    return f(_compact_kernel)(keep_flat, tq, pms_flat)


def _decode_kernel(params_ref, idx_ref, score_ref, ratio_ref,
                   mbig_ref, f_refs, l0_ref, l1_ref):
    p = params_ref[0][:, :P]
    pa = jnp.concatenate([p, jnp.ones((KPAD, 1), jnp.float32)], axis=1)
    m = lax.dot_general(mbig_ref[...], pa, (((1,), (1,)), ((), ())),
                        precision=lax.Precision.HIGHEST,
                        preferred_element_type=jnp.float32)
    s_row = m[0:1]
    r00, r01, r02 = m[1:2], m[2:3], m[3:4]
    r10, r11, r12 = m[4:5], m[5:6], m[6:7]
    r20, r21, r22 = m[7:8], m[8:9], m[9:10]
    v0 = m[10:10 + KPTS]
    v1 = m[10 + KPTS:10 + 2 * KPTS]
    v2 = m[10 + 2 * KPTS:10 + 3 * KPTS]
    xc = s_row * (v0 * r00 + v1 * r01 + v2 * r02)
    yc = s_row * (v0 * r10 + v1 * r11 + v2 * r12)
    idxv = idx_ref[0]
    bb = pl.program_id(0)
    ys = (idxv // W).astype(jnp.float32) * ratio_ref[bb, 0]
    xs = (idxv % W).astype(jnp.float32) * ratio_ref[bb, 1]
    ln0 = yc + ys
    ln1 = xc + xs
    tl0 = jnp.min(ln0, axis=0, keepdims=True)
    tl1 = jnp.min(ln1, axis=0, keepdims=True)
    br0 = jnp.max(ln0, axis=0, keepdims=True)
    br1 = jnp.max(ln1, axis=0, keepdims=True)
    area = (br0 - tl0) * (br1 - tl1)
    score = score_ref[0]
    fields = [tl0, tl1, br0, br1, area, score, r20, r21, r22, r10, r00]
    for ref, val in zip(f_refs, fields):
        ref[0] = val
    l0_ref[0] = ln0
    l1_ref[0] = ln1


def _decode_call(params, idx3, score3, ratio, mbig):
    nf = 11
    out_shape = ([jax.ShapeDtypeStruct((B, 1, KPAD), jnp.float32)] * nf
                 + [jax.ShapeDtypeStruct((B, KPTS, KPAD), jnp.float32)] * 2)
    row_spec = pl.BlockSpec((1, 1, KPAD), lambda b: (b, 0, 0))
    outs = pl.pallas_call(
        lambda pr, ir, sr, rr, mr, *os: _decode_kernel(
            pr, ir, sr, rr, mr, os[:nf], os[nf], os[nf + 1]),
        grid=(B,),
        in_specs=[
            pl.BlockSpec((1, KPAD, PPAD), lambda b: (b, 0, 0)),
            pl.BlockSpec((1, 1, KPAD), lambda b: (b, 0, 0)),
            pl.BlockSpec((1, 1, KPAD), lambda b: (b, 0, 0)),
            pl.BlockSpec(memory_space=pltpu.SMEM, block_shape=(B, 2),
                         index_map=lambda b: (0, 0)),
            pl.BlockSpec((214, P + 1), lambda b: (0, 0)),
        ],
        out_specs=[row_spec] * nf + [
            pl.BlockSpec((1, KPTS, KPAD), lambda b: (b, 0, 0))] * 2,
        out_shape=out_shape,
        interpret=_INTERPRET,
    )(params, idx3, score3, ratio, mbig)
    return outs[:nf], outs[nf], outs[nf + 1]


def _nms_kernel(tl0_ref, tl1_ref, br0_ref, br1_ref, area_ref, score_ref,
                r20_ref, r21_ref, r22_ref, r10_ref, r00_ref,
                l0_ref, l1_ref,
                self_ref, sl0_ref, sl1_ref, pose_ref, oh_ref):
    lane = lax.broadcasted_iota(jnp.int32, (B, KPAD), 1)
    score = score_ref[...]
    tl0 = tl0_ref[...]
    tl1 = tl1_ref[...]
    br0 = br0_ref[...]
    br1 = br1_ref[...]
    area = area_ref[...]
    work0 = jnp.where(lane < K, score, NEG)
    m0 = jnp.max(work0, axis=1, keepdims=True)
    best0 = jnp.min(jnp.where(work0 == m0, lane, KPAD), axis=1, keepdims=True)

    def step(r, work):
        m = jnp.max(work, axis=1, keepdims=True)
        alive = m > (0.5 * NEG)
        bm = jnp.where(work == m, lane, KPAD)
        best = jnp.min(bm, axis=1, keepdims=True)
        best = jnp.where(alive, best, best0)
        oh = lane == best
        ohf = oh.astype(jnp.float32)
        oh_ref[r] = ohf

        def ext(f):
            return jnp.sum(jnp.where(oh, f, 0.0), axis=1, keepdims=True)

        btl0, btl1, bbr0, bbr1, barea = (
            ext(tl0), ext(tl1), ext(br0), ext(br1), ext(area))
        yy1 = jnp.maximum(btl0, tl0)
        xx1 = jnp.maximum(btl1, tl1)
        yy2 = jnp.minimum(bbr0, br0)
        xx2 = jnp.minimum(bbr1, br1)
        inter = jnp.maximum(yy2 - yy1, 0.0) * jnp.maximum(xx2 - xx1, 0.0)
        iou = inter / (barea + area - inter + 1e-8)
        sup = (iou > IOU_THR) | oh
        return jnp.where(sup, NEG, work)

    lax.fori_loop(0, NOBJ, step, work0)

    deg = jnp.float32(180.0 / np.pi)
    for b in range(B):
        ohb = oh_ref[:, b]
        dn = (((1,), (1,)), ((), ()))
        sl0 = lax.dot_general(ohb, l0_ref[b], dn,
                              precision=lax.Precision.HIGHEST,
                              preferred_element_type=jnp.float32)
        sl1 = lax.dot_general(ohb, l1_ref[b], dn,
                              precision=lax.Precision.HIGHEST,
                              preferred_element_type=jnp.float32)
        sl0_ref[b] = sl0
        sl1_ref[b] = sl1
        fstack = jnp.concatenate(
            [tl0_ref[b:b + 1], tl1_ref[b:b + 1], br0_ref[b:b + 1],
             br1_ref[b:b + 1], score_ref[b:b + 1],
             r20_ref[b:b + 1], r21_ref[b:b + 1], r22_ref[b:b + 1],
             r10_ref[b:b + 1], r00_ref[b:b + 1],
             jnp.zeros((6, KPAD), jnp.float32)], axis=0)
        fs = lax.dot_general(ohb, fstack, dn,
                             precision=lax.Precision.HIGHEST,
                             preferred_element_type=jnp.float32)
        self_ref[b] = fs
        sy = jnp.clip(-fs[:, 5:6], -0.999, 0.999)
        yaw = jnp.arctan2(sy, jnp.sqrt(1.0 - sy * sy)) * deg
        cy = jnp.cos(yaw)
        cy = jnp.where(jnp.abs(cy) < 1e-6, 1e-6, cy)
        pitch = jnp.arctan2(fs[:, 6:7] / cy, fs[:, 7:8] / cy) * deg
        roll = jnp.arctan2(fs[:, 8:9] / cy, fs[:, 9:10] / cy) * deg
        pose_ref[b] = jnp.concatenate(
            [pitch, yaw, roll, jnp.zeros((NOBJ, 13), jnp.float32)], axis=1)


def _nms_call(frows, l0, l1):
    full2 = pl.BlockSpec((B, KPAD), lambda: (0, 0))
    full3 = pl.BlockSpec((B, KPTS, KPAD), lambda: (0, 0, 0))
    out_shape = [
        jax.ShapeDtypeStruct((B, NOBJ, 16), jnp.float32),
        jax.ShapeDtypeStruct((B, NOBJ, KPTS), jnp.float32),
        jax.ShapeDtypeStruct((B, NOBJ, KPTS), jnp.float32),
        jax.ShapeDtypeStruct((B, NOBJ, 16), jnp.float32),
    ]
    return pl.pallas_call(
        _nms_kernel,
        in_specs=[full2] * 11 + [full3] * 2,
        out_specs=[
            pl.BlockSpec((B, NOBJ, 16), lambda: (0, 0, 0)),
            pl.BlockSpec((B, NOBJ, KPTS), lambda: (0, 0, 0)),
            pl.BlockSpec((B, NOBJ, KPTS), lambda: (0, 0, 0)),
            pl.BlockSpec((B, NOBJ, 16), lambda: (0, 0, 0)),
        ],
        out_shape=out_shape,
        scratch_shapes=[pltpu.VMEM((NOBJ, B, KPAD), jnp.float32)],
        interpret=_INTERPRET,
    )(*frows, l0, l1)


def _build_mbig(pms_stats, u_base, shp_base, exp_base):
    mean, std = pms_stats[0], pms_stats[1]
    basis = jnp.concatenate([shp_base, exp_base], axis=1)
    rows = []
    sel_cols = [0, 1, 2, 3, 5, 6, 7, 9, 10, 11]
    eye = jnp.eye(P, dtype=jnp.float32)
    for c in sel_cols:
        row = jnp.concatenate([eye[c] * std[c],
                               jnp.array([mean[c]], jnp.float32)])
        rows.append(row)
    sel_part = jnp.stack(rows)
    vrows = []
    for j in range(3):
        bj = basis[j::3]
        wj = bj * std[12:]
        bias = bj @ mean[12:] + u_base[j::3, 0]
        left = jnp.zeros((KPTS, 12), jnp.float32)
        vrows.append(jnp.concatenate([left, wj, bias[:, None]], axis=1))
    return jnp.concatenate([sel_part] + vrows, axis=0)


def kernel(hms, pms_map, origin_shapes, pms_stats, u_base, shp_base, exp_base):
    hm = hms[..., 0]
    keep = _peak_call(hm)
    keep_flat = keep.reshape(B, HW)
    tq = _search_call(keep_flat)
    pms_pad = jnp.pad(pms_map.reshape(B, HW, P), ((0, 0), (0, 0), (0, PPAD - P)))
    idx, score, params = _compact_call(keep_flat, tq, pms_pad)
    ratio = origin_shapes / jnp.array([float(H), float(W)], jnp.float32)
    mbig = _build_mbig(pms_stats, u_base, shp_base, exp_base)
    idx3 = idx[:, None, :]
    score3 = score[:, None, :]
    frows, l0, l1 = _decode_call(params, idx3, score3, ratio, mbig)
    frows2 = [f.reshape(B, KPAD) for f in frows]
    self_, sl0, sl1, pose16 = _nms_call(frows2, l0, l1)
    out_bboxes = jnp.concatenate(
        [self_[:, :, 0:5], jnp.zeros((B, NOBJ, 1), jnp.float32)], axis=-1)
    out_lnmks = jnp.stack([sl0, sl1], axis=-1)
    pose = pose16[:, :, 0:3]
    return out_bboxes, out_lnmks, pose
